# Initial kernel scaffold; baseline (speedup 1.0000x reference)
import jax
import jax.numpy as jnp
from jax.experimental import pallas as pl


def kernel(x, W_d, W_f, W_pos, b_pos):
    raise NotImplementedError("write your pallas kernel here")


# speedup vs baseline: 4.3063x; 4.3063x over previous
import functools

import jax
import jax.numpy as jnp
from jax import lax
from jax.experimental import pallas as pl
from jax.experimental.pallas import tpu as pltpu

_R2 = 0.25
_K = 32
_N = 2048
_M = 256
_B = 4
_TP = 8
_DIM = 128
_ROWS = _TP * _B


def _fps_body(pts_ref, anch_ref):
    px = pts_ref[0]
    py = pts_ref[1]
    pz = pts_ref[2]
    iota = lax.broadcasted_iota(jnp.int32, (_ROWS, _N), 1)

    def last_coords(nxt):
        oh = iota == nxt
        lx = jnp.sum(jnp.where(oh, px, 0.0), axis=1, keepdims=True)
        ly = jnp.sum(jnp.where(oh, py, 0.0), axis=1, keepdims=True)
        lz = jnp.sum(jnp.where(oh, pz, 0.0), axis=1, keepdims=True)
        return lx, ly, lz

    iota_m = lax.broadcasted_iota(jnp.int32, (_ROWS, _M), 1)

    def body(i, carry):
        dists, nxt, axv, ayv, azv = carry
        lx, ly, lz = last_coords(nxt)
        sel = iota_m == (i - 1)
        axv = jnp.where(sel, lx, axv)
        ayv = jnp.where(sel, ly, ayv)
        azv = jnp.where(sel, lz, azv)
        dx = px - lx
        dy = py - ly
        dz = pz - lz
        d = (dx * dx + dy * dy) + dz * dz
        dists = jnp.minimum(dists, d)
        maxv = jnp.max(dists, axis=1, keepdims=True)
        cand = jnp.where(dists == maxv, iota, _N)
        nxt = jnp.min(cand, axis=1, keepdims=True)
        return dists, nxt, axv, ayv, azv

    dists0 = jnp.full((_ROWS, _N), 1e10, dtype=jnp.float32)
    nxt0 = jnp.zeros((_ROWS, 1), dtype=jnp.int32)
    a0 = jnp.zeros((_ROWS, _M), dtype=jnp.float32)
    _, nxt, axv, ayv, azv = lax.fori_loop(
        1, _M, body, (dists0, nxt0, a0, a0, a0))
    lx, ly, lz = last_coords(nxt)
    sel = iota_m == (_M - 1)
    anch_ref[0] = jnp.where(sel, lx, axv)
    anch_ref[1] = jnp.where(sel, ly, ayv)
    anch_ref[2] = jnp.where(sel, lz, azv)


def _enc_body(pts_ref, ax_ref, ay_ref, az_ref, wg_ref, wdelta_ref, cvec_ref,
              out_ref, g_ref, take_ref):
    j = pl.program_id(0)
    axv = ax_ref[0, 0][:, None]
    ayv = ay_ref[0, 0][:, None]
    azv = az_ref[0, 0][:, None]

    utri = (lax.broadcasted_iota(jnp.int32, (_DIM, _DIM), 0)
            <= lax.broadcasted_iota(jnp.int32, (_DIM, _DIM), 1)
            ).astype(jnp.float32)
    stri16 = (lax.broadcasted_iota(jnp.int32, (16, 16), 0)
              < lax.broadcasted_iota(jnp.int32, (16, 16), 1)
              ).astype(jnp.float32)

    acc = jnp.full((_M, _DIM), -1e30, dtype=jnp.float32)
    for k in range(3):
        p = pts_ref[0, k, 0]
        px = p[:, 0][None, :]
        py = p[:, 1][None, :]
        pz = p[:, 2][None, :]
        g = jnp.dot(p, wg_ref[:3, :], preferred_element_type=jnp.float32)
        g = g + p[:, 2][:, None] * wg_ref[3, :][None, :]
        g = g + jnp.float32(k - 1) * cvec_ref[0, :][None, :]
        dx = axv - px
        dy = ayv - py
        dz = azv - pz
        d2 = (dx * dx + dy * dy) + dz * dz
        mask = (d2 < _R2).astype(jnp.float32)
        m3 = mask.reshape(_M * 16, _DIM)
        incl = jnp.dot(m3, utri, preferred_element_type=jnp.float32)
        incl = incl.reshape(_M, 16, _DIM)
        tot = incl[:, :, _DIM - 1]
        base = jnp.dot(tot, stri16, preferred_element_type=jnp.float32)
        inclg = incl + base[:, :, None]
        take = jnp.where(
            (mask.reshape(_M, 16, _DIM) > 0.0) & (inclg <= jnp.float32(_K)),
            1.0, 0.0)
        none_row = (tot[:, 15] + base[:, 15])[:, None] == 0.0
        acc = jnp.maximum(acc, jnp.where(none_row, g[0, :][None, :], -1e30))

        g_ref[...] = g
        for blk in range(16):
            take_ref[blk] = take[:, blk, :]

        def blk_body(blk, a):
            gb = g_ref[pl.ds(blk * _DIM, _DIM), :]
            tb = take_ref[blk]
            for s in range(16):
                t8 = tb[:, s * 8:(s + 1) * 8][:, :, None] > 0.0
                g8 = gb[s * 8:(s + 1) * 8, :][None, :, :]
                vals = jnp.where(t8, g8, -1e30)
                a = jnp.maximum(a, jnp.max(vals, axis=1))
            return a

        acc = lax.fori_loop(0, 16, blk_body, acc)

    a3 = jnp.concatenate([axv, ayv, azv], axis=1)
    corr = jnp.dot(a3, wdelta_ref[...], preferred_element_type=jnp.float32)
    ts = (j + 1).astype(jnp.float32)
    outv = acc + corr + ts * cvec_ref[1, :][None, :] + cvec_ref[2, :][None, :]
    out_ref[0, 0] = jnp.maximum(outv, 0.0)


@jax.jit
def kernel(x, W_d, W_f, W_pos, b_pos):
    xf = x[:, ::2, :, :3]
    pts_fps = xf.transpose(3, 1, 0, 2).reshape(3, _ROWS, _N)
    anchors = pl.pallas_call(
        _fps_body,
        out_shape=jax.ShapeDtypeStruct((3, _ROWS, _M), jnp.float32),
    )(pts_fps)
    ax = anchors[0].reshape(_ROWS, 1, _M)
    ay = anchors[1].reshape(_ROWS, 1, _M)
    az = anchors[2].reshape(_ROWS, 1, _M)

    fids = [[max(2 * j - 1, 0), 2 * j, min(2 * j + 1, 15)] for j in range(_TP)]
    fids = jnp.asarray(fids, dtype=jnp.int32)
    pts = x[:, fids, :, :3]
    pts = pts.transpose(1, 2, 0, 3, 4)

    wg = jnp.concatenate([W_d[:, :3].T, W_f[:, 0][None, :]], axis=0)
    wdelta = (W_pos[:, :3] - W_d[:, :3]).T
    cvec = jnp.stack([W_d[:, 3], W_pos[:, 3], b_pos,
                      jnp.zeros_like(b_pos)], axis=0)

    out = pl.pallas_call(
        _enc_body,
        grid=(_TP, _B),
        in_specs=[
            pl.BlockSpec((1, 3, 1, _N, 3), lambda j, b: (j, 0, b, 0, 0)),
            pl.BlockSpec((1, 1, _M), lambda j, b: (j * _B + b, 0, 0)),
            pl.BlockSpec((1, 1, _M), lambda j, b: (j * _B + b, 0, 0)),
            pl.BlockSpec((1, 1, _M), lambda j, b: (j * _B + b, 0, 0)),
            pl.BlockSpec((4, _DIM), lambda j, b: (0, 0)),
            pl.BlockSpec((3, _DIM), lambda j, b: (0, 0)),
            pl.BlockSpec((4, _DIM), lambda j, b: (0, 0)),
        ],
        out_specs=pl.BlockSpec((1, 1, _M, _DIM), lambda j, b: (j, b, 0, 0)),
        out_shape=jax.ShapeDtypeStruct((_TP, _B, _M, _DIM), jnp.float32),
        scratch_shapes=[
            pltpu.VMEM((_N, _DIM), jnp.float32),
            pltpu.VMEM((16, _M, _DIM), jnp.float32),
        ],
    )(pts, ax, ay, az, wg, wdelta, cvec)

    return out.transpose(1, 0, 2, 3).reshape(_B, _TP * _M, _DIM)


# speedup vs baseline: 8.9836x; 2.0861x over previous
import functools

import jax
import jax.numpy as jnp
from jax import lax
from jax.experimental import pallas as pl
from jax.experimental.pallas import tpu as pltpu
from jax.experimental.pallas import tpu_sc as plsc

_R2 = 0.25
_K = 32
_N = 2048
_M = 256
_B = 4
_TP = 8
_DIM = 128
_ROWS = _TP * _B
_NP = _TP * 3
_NTASK = _NP * _B * _M
_NWORK = 32
_TPW = _NTASK // _NWORK


def _fps_body(pts_ref, anch_ref):
    px = pts_ref[0]
    py = pts_ref[1]
    pz = pts_ref[2]
    iota = lax.broadcasted_iota(jnp.int32, (_ROWS, _N), 1)
    iota_m = lax.broadcasted_iota(jnp.int32, (_ROWS, _M), 1)

    def last_coords(nxt):
        oh = iota == nxt
        lx = jnp.sum(jnp.where(oh, px, 0.0), axis=1, keepdims=True)
        ly = jnp.sum(jnp.where(oh, py, 0.0), axis=1, keepdims=True)
        lz = jnp.sum(jnp.where(oh, pz, 0.0), axis=1, keepdims=True)
        return lx, ly, lz

    def body(i, carry):
        dists, nxt, axv, ayv, azv = carry
        lx, ly, lz = last_coords(nxt)
        sel = iota_m == (i - 1)
        axv = jnp.where(sel, lx, axv)
        ayv = jnp.where(sel, ly, ayv)
        azv = jnp.where(sel, lz, azv)
        dx = px - lx
        dy = py - ly
        dz = pz - lz
        d = (dx * dx + dy * dy) + dz * dz
        dists = jnp.minimum(dists, d)
        maxv = jnp.max(dists, axis=1, keepdims=True)
        cand = jnp.where(dists == maxv, iota, _N)
        nxt = jnp.min(cand, axis=1, keepdims=True)
        return dists, nxt, axv, ayv, azv

    dists0 = jnp.full((_ROWS, _N), 1e10, dtype=jnp.float32)
    nxt0 = jnp.zeros((_ROWS, 1), dtype=jnp.int32)
    a0 = jnp.zeros((_ROWS, _M), dtype=jnp.float32)
    _, nxt, axv, ayv, azv = lax.fori_loop(
        1, _M, body, (dists0, nxt0, a0, a0, a0))
    lx, ly, lz = last_coords(nxt)
    sel = iota_m == (_M - 1)
    anch_ref[0] = jnp.where(sel, lx, axv)
    anch_ref[1] = jnp.where(sel, ly, ayv)
    anch_ref[2] = jnp.where(sel, lz, azv)


def _stage2_body(pts_ref, ax_ref, ay_ref, az_ref, wg_ref, cvec_ref,
                 g_ref, idx_ref):
    p_id = pl.program_id(0)
    b_id = pl.program_id(1)
    tapc = (p_id % 3 - 1).astype(jnp.float32)
    rowbase = ((p_id * _B + b_id) * _N).astype(jnp.float32)
    axv = ax_ref[0, 0][:, None]
    ayv = ay_ref[0, 0][:, None]
    azv = az_ref[0, 0][:, None]

    pts = pts_ref[0, 0]
    px = pts[:, 0][None, :]
    py = pts[:, 1][None, :]
    pz = pts[:, 2][None, :]
    g = jnp.dot(pts, wg_ref[:3, :], preferred_element_type=jnp.float32)
    g = g + pts[:, 2][:, None] * wg_ref[3, :][None, :]
    g = g + tapc * cvec_ref[0, :][None, :]
    g_ref[0, 0] = g

    dx = axv - px
    dy = ayv - py
    dz = azv - pz
    d2 = (dx * dx + dy * dy) + dz * dz
    mask = (d2 < _R2).astype(jnp.float32)
    utri = (lax.broadcasted_iota(jnp.int32, (_DIM, _DIM), 0)
            <= lax.broadcasted_iota(jnp.int32, (_DIM, _DIM), 1)
            ).astype(jnp.float32)
    stri16 = (lax.broadcasted_iota(jnp.int32, (16, 16), 0)
              < lax.broadcasted_iota(jnp.int32, (16, 16), 1)
              ).astype(jnp.float32)
    m3 = mask.reshape(_M * 16, _DIM)
    incl = jnp.dot(m3, utri, preferred_element_type=jnp.float32)
    incl = incl.reshape(_M, 16, _DIM)
    tot = incl[:, :, _DIM - 1]
    base = jnp.dot(tot, stri16, preferred_element_type=jnp.float32)
    inclg = incl + base[:, :, None]
    take3 = (mask.reshape(_M, 16, _DIM) > 0.0) & (inclg <= jnp.float32(_K))

    bi = lax.broadcasted_iota(jnp.int32, (_M, 16, _DIM), 1)
    li = lax.broadcasted_iota(jnp.int32, (_M, 16, _DIM), 2)
    n3 = (bi * _DIM + li).astype(jnp.float32)
    ptn = jnp.where(take3, n3, 0.0)
    rk = jnp.where(take3, inclg, 0.0)

    slots = []
    for k in range(_K):
        v = jnp.where(rk == jnp.float32(k + 1), ptn, 0.0)
        slots.append(jnp.sum(v, axis=(1, 2))[:, None])
    idxmat = jnp.concatenate(slots, axis=1)
    total = base[:, 15] + tot[:, 15]
    cntm = jnp.minimum(total, jnp.float32(_K))[:, None]
    kio = lax.broadcasted_iota(jnp.int32, (_M, _K), 1).astype(jnp.float32)
    idxmat = jnp.where(kio < cntm, idxmat, idxmat[:, 0:1])
    idx_ref[0, 0] = (idxmat + rowbase).astype(jnp.int32)


def _sc_body(idx_hbm, g_hbm, out_hbm, idxv, rows, outv, sem):
    cid = lax.axis_index("c")
    sid = lax.axis_index("s")
    wid = sid * 2 + cid
    base_task = wid * _TPW

    def task(i, _):
        t = base_task + i
        pltpu.sync_copy(idx_hbm.at[t], idxv)
        pltpu.async_copy(g_hbm.at[idxv], rows, sem).wait()
        for h in range(8):
            acc = rows[0, pl.ds(h * 16, 16)]
            for r in range(1, _K):
                acc = jnp.maximum(acc, rows[r, pl.ds(h * 16, 16)])
            outv[pl.ds(h * 16, 16)] = acc
        pltpu.sync_copy(outv, out_hbm.at[t])
        return 0

    lax.fori_loop(0, _TPW, task, 0)


def _epi_body(mx_ref, ax_ref, ay_ref, az_ref, wdelta_ref, cvec_ref,
              out_ref):
    j = pl.program_id(0)
    acc = jnp.maximum(jnp.maximum(mx_ref[0, 0, 0], mx_ref[0, 1, 0]),
                      mx_ref[0, 2, 0])
    axv = ax_ref[0, 0][:, None]
    ayv = ay_ref[0, 0][:, None]
    azv = az_ref[0, 0][:, None]
    a3 = jnp.concatenate([axv, ayv, azv], axis=1)
    corr = jnp.dot(a3, wdelta_ref[...], preferred_element_type=jnp.float32)
    ts = (j + 1).astype(jnp.float32)
    outv = acc + corr + ts * cvec_ref[1, :][None, :] + cvec_ref[2, :][None, :]
    out_ref[0, 0] = jnp.maximum(outv, 0.0)


@jax.jit
def kernel(x, W_d, W_f, W_pos, b_pos):
    xf = x[:, ::2, :, :3]
    pts_fps = xf.transpose(3, 1, 0, 2).reshape(3, _ROWS, _N)
    anchors = pl.pallas_call(
        _fps_body,
        out_shape=jax.ShapeDtypeStruct((3, _ROWS, _M), jnp.float32),
    )(pts_fps)
    ax = anchors[0].reshape(_ROWS, 1, _M)
    ay = anchors[1].reshape(_ROWS, 1, _M)
    az = anchors[2].reshape(_ROWS, 1, _M)

    fids = [[max(2 * j - 1, 0), 2 * j, min(2 * j + 1, 15)] for j in range(_TP)]
    fids = jnp.asarray(fids, dtype=jnp.int32)
    pts = x[:, fids, :, :3]
    pts = pts.transpose(1, 2, 0, 3, 4).reshape(_NP, _B, _N, 3)

    wg = jnp.concatenate([W_d[:, :3].T, W_f[:, 0][None, :]], axis=0)
    wdelta = (W_pos[:, :3] - W_d[:, :3]).T
    cvec = jnp.stack([W_d[:, 3], W_pos[:, 3], b_pos,
                      jnp.zeros_like(b_pos)], axis=0)

    anch_spec = pl.BlockSpec((1, 1, _M), lambda p, b: ((p // 3) * _B + b, 0, 0))
    g_all, idx = pl.pallas_call(
        _stage2_body,
        grid=(_NP, _B),
        in_specs=[
            pl.BlockSpec((1, 1, _N, 3), lambda p, b: (p, b, 0, 0)),
            anch_spec,
            anch_spec,
            anch_spec,
            pl.BlockSpec((4, _DIM), lambda p, b: (0, 0)),
            pl.BlockSpec((4, _DIM), lambda p, b: (0, 0)),
        ],
        out_specs=[
            pl.BlockSpec((1, 1, _N, _DIM), lambda p, b: (p, b, 0, 0)),
            pl.BlockSpec((1, 1, _M, _K), lambda p, b: (p, b, 0, 0)),
        ],
        out_shape=[
            jax.ShapeDtypeStruct((_NP, _B, _N, _DIM), jnp.float32),
            jax.ShapeDtypeStruct((_NP, _B, _M, _K), jnp.int32),
        ],
    )(pts, ax, ay, az, wg, cvec)

    idx_flat = idx.reshape(_NTASK, _K)
    g_flat = g_all.reshape(_NP * _B * _N, _DIM)

    mesh = plsc.VectorSubcoreMesh(core_axis_name="c", subcore_axis_name="s")
    maxg = pl.kernel(
        _sc_body,
        mesh=mesh,
        out_type=jax.ShapeDtypeStruct((_NTASK, _DIM), jnp.float32),
        scratch_types=[
            pltpu.VMEM((_K,), jnp.int32),
            pltpu.VMEM((_K, _DIM), jnp.float32),
            pltpu.VMEM((_DIM,), jnp.float32),
            pltpu.SemaphoreType.DMA,
        ],
    )(idx_flat, g_flat)

    mx = maxg.reshape(_TP, 3, _B, _M, _DIM)
    epi_anch = pl.BlockSpec((1, 1, _M), lambda j, b: (j * _B + b, 0, 0))
    out = pl.pallas_call(
        _epi_body,
        grid=(_TP, _B),
        in_specs=[
            pl.BlockSpec((1, 3, 1, _M, _DIM), lambda j, b: (j, 0, b, 0, 0)),
            epi_anch,
            epi_anch,
            epi_anch,
            pl.BlockSpec((3, _DIM), lambda j, b: (0, 0)),
            pl.BlockSpec((4, _DIM), lambda j, b: (0, 0)),
        ],
        out_specs=pl.BlockSpec((1, 1, _M, _DIM), lambda j, b: (j, b, 0, 0)),
        out_shape=jax.ShapeDtypeStruct((_TP, _B, _M, _DIM), jnp.float32),
    )(mx, ax, ay, az, wdelta, cvec)

    return out.transpose(1, 0, 2, 3).reshape(_B, _TP * _M, _DIM)


# speedup vs baseline: 10.4167x; 1.1595x over previous
import jax
import jax.numpy as jnp
from jax import lax
from jax.experimental import pallas as pl
from jax.experimental.pallas import tpu as pltpu
from jax.experimental.pallas import tpu_sc as plsc

_R2 = 0.25
_K = 32
_N = 2048
_M = 256
_B = 4
_TP = 8
_DIM = 128
_ROWS = _TP * _B
_NP = _TP * 3
_NTASK = _NP * _B * _M
_NWORK = 32
_TPW = _NTASK // _NWORK


def _fps_body(pts_ref, anch_ref):
    px = pts_ref[0]
    py = pts_ref[1]
    pz = pts_ref[2]
    iota = lax.broadcasted_iota(jnp.int32, (_ROWS, _N), 1)
    iota_m = lax.broadcasted_iota(jnp.int32, (_ROWS, _M), 1)

    def last_coords(nxt):
        oh = iota == nxt
        lx = jnp.sum(jnp.where(oh, px, 0.0), axis=1, keepdims=True)
        ly = jnp.sum(jnp.where(oh, py, 0.0), axis=1, keepdims=True)
        lz = jnp.sum(jnp.where(oh, pz, 0.0), axis=1, keepdims=True)
        return lx, ly, lz

    def body(i, carry):
        dists, nxt, axv, ayv, azv = carry
        lx, ly, lz = last_coords(nxt)
        sel = iota_m == (i - 1)
        axv = jnp.where(sel, lx, axv)
        ayv = jnp.where(sel, ly, ayv)
        azv = jnp.where(sel, lz, azv)
        dx = px - lx
        dy = py - ly
        dz = pz - lz
        d = (dx * dx + dy * dy) + dz * dz
        dists = jnp.minimum(dists, d)
        maxv = jnp.max(dists, axis=1, keepdims=True)
        cand = jnp.where(dists == maxv, iota, _N)
        nxt = jnp.min(cand, axis=1, keepdims=True)
        return dists, nxt, axv, ayv, azv

    dists0 = jnp.full((_ROWS, _N), 1e10, dtype=jnp.float32)
    nxt0 = jnp.zeros((_ROWS, 1), dtype=jnp.int32)
    a0 = jnp.zeros((_ROWS, _M), dtype=jnp.float32)
    _, nxt, axv, ayv, azv = lax.fori_loop(
        1, _M, body, (dists0, nxt0, a0, a0, a0))
    lx, ly, lz = last_coords(nxt)
    sel = iota_m == (_M - 1)
    anch_ref[0] = jnp.where(sel, lx, axv)
    anch_ref[1] = jnp.where(sel, ly, ayv)
    anch_ref[2] = jnp.where(sel, lz, azv)


def _stage2_body(pts_ref, ax_ref, ay_ref, az_ref, wg_ref, cvec_ref,
                 g_ref, idx_ref):
    p_id = pl.program_id(0)
    b_id = pl.program_id(1)
    tapc = (p_id % 3 - 1).astype(jnp.float32)
    rowbase = ((p_id * _B + b_id) * _N).astype(jnp.float32)
    axv = ax_ref[0, 0][:, None]
    ayv = ay_ref[0, 0][:, None]
    azv = az_ref[0, 0][:, None]

    pts = pts_ref[0, 0]
    px = pts[:, 0][None, :]
    py = pts[:, 1][None, :]
    pz = pts[:, 2][None, :]
    g = jnp.dot(pts, wg_ref[:3, :], preferred_element_type=jnp.float32)
    g = g + pts[:, 2][:, None] * wg_ref[3, :][None, :]
    g = g + tapc * cvec_ref[0, :][None, :]
    g_ref[0, 0] = g

    dx = axv - px
    dy = ayv - py
    dz = azv - pz
    d2 = (dx * dx + dy * dy) + dz * dz
    mask = (d2 < _R2).astype(jnp.float32)
    utri = (lax.broadcasted_iota(jnp.int32, (_DIM, _DIM), 0)
            <= lax.broadcasted_iota(jnp.int32, (_DIM, _DIM), 1)
            ).astype(jnp.float32)
    stri16 = (lax.broadcasted_iota(jnp.int32, (16, 16), 0)
              < lax.broadcasted_iota(jnp.int32, (16, 16), 1)
              ).astype(jnp.float32)
    m3 = mask.reshape(_M * 16, _DIM)
    incl = jnp.dot(m3, utri, preferred_element_type=jnp.float32)
    incl = incl.reshape(_M, 16, _DIM)
    tot = incl[:, :, _DIM - 1]
    base = jnp.dot(tot, stri16, preferred_element_type=jnp.float32)
    inclg = incl + base[:, :, None]
    take3 = (mask.reshape(_M, 16, _DIM) > 0.0) & (inclg <= jnp.float32(_K))

    bi = lax.broadcasted_iota(jnp.int32, (_M, 16, _DIM), 1)
    li = lax.broadcasted_iota(jnp.int32, (_M, 16, _DIM), 2)
    n3 = (bi * _DIM + li).astype(jnp.float32)
    ptn = jnp.where(take3, n3, 0.0)
    rk = jnp.where(take3, inclg, 0.0)

    slots = []
    for k in range(_K):
        v = jnp.where(rk == jnp.float32(k + 1), ptn, 0.0)
        slots.append(jnp.sum(v, axis=(1, 2))[:, None])
    idxmat = jnp.concatenate(slots, axis=1)
    total = base[:, 15] + tot[:, 15]
    cntm = jnp.minimum(total, jnp.float32(_K))[:, None]
    kio = lax.broadcasted_iota(jnp.int32, (_M, _K), 1).astype(jnp.float32)
    idxmat = jnp.where(kio < cntm, idxmat, idxmat[:, 0:1])
    idx_ref[0, 0] = (idxmat + rowbase).astype(jnp.int32)


def _sc_body(idx_hbm, g_hbm, out_hbm, idxv0, idxv1, rows0, rows1, outv0,
             outv1, gsem0, gsem1, osem0, osem1):
    cid = lax.axis_index("c")
    sid = lax.axis_index("s")
    wid = sid * 2 + cid
    base_task = wid * _TPW

    def _maxrows(rows, outv):
        for h in range(8):
            acc = rows[0, pl.ds(h * 16, 16)]
            for r in range(1, _K):
                acc = jnp.maximum(acc, rows[r, pl.ds(h * 16, 16)])
            outv[pl.ds(h * 16, 16)] = acc

    pltpu.sync_copy(idx_hbm.at[base_task], idxv0)
    pltpu.async_copy(g_hbm.at[idxv0], rows0, gsem0)
    pltpu.async_copy(outv0, out_hbm.at[base_task], osem0)
    pltpu.async_copy(outv1, out_hbm.at[base_task + 1], osem1)

    def pair(ip, _):
        t0 = base_task + 2 * ip
        pltpu.sync_copy(idx_hbm.at[t0 + 1], idxv1)
        pltpu.make_async_copy(g_hbm.at[idxv0], rows0, gsem0).wait()
        pltpu.async_copy(g_hbm.at[idxv1], rows1, gsem1)
        pltpu.make_async_copy(outv0, out_hbm.at[t0], osem0).wait()
        _maxrows(rows0, outv0)
        pltpu.async_copy(outv0, out_hbm.at[t0], osem0)
        tnx = jnp.minimum(t0 + 2, _NTASK - 1)
        pltpu.sync_copy(idx_hbm.at[tnx], idxv0)
        pltpu.make_async_copy(g_hbm.at[idxv1], rows1, gsem1).wait()
        pltpu.async_copy(g_hbm.at[idxv0], rows0, gsem0)
        pltpu.make_async_copy(outv1, out_hbm.at[t0 + 1], osem1).wait()
        _maxrows(rows1, outv1)
        pltpu.async_copy(outv1, out_hbm.at[t0 + 1], osem1)
        return 0

    lax.fori_loop(0, _TPW // 2, pair, 0)
    pltpu.make_async_copy(g_hbm.at[idxv0], rows0, gsem0).wait()
    pltpu.make_async_copy(outv0, out_hbm.at[base_task], osem0).wait()
    pltpu.make_async_copy(outv1, out_hbm.at[base_task], osem1).wait()


def _epi_body(mx_ref, ax_ref, ay_ref, az_ref, wdelta_ref, cvec_ref,
              out_ref):
    j = pl.program_id(0)
    acc = jnp.maximum(jnp.maximum(mx_ref[0, 0, 0], mx_ref[0, 1, 0]),
                      mx_ref[0, 2, 0])
    axv = ax_ref[0, 0][:, None]
    ayv = ay_ref[0, 0][:, None]
    azv = az_ref[0, 0][:, None]
    a3 = jnp.concatenate([axv, ayv, azv], axis=1)
    corr = jnp.dot(a3, wdelta_ref[...], preferred_element_type=jnp.float32)
    ts = (j + 1).astype(jnp.float32)
    outv = acc + corr + ts * cvec_ref[1, :][None, :] + cvec_ref[2, :][None, :]
    out_ref[0, 0] = jnp.maximum(outv, 0.0)


@jax.jit
def kernel(x, W_d, W_f, W_pos, b_pos):
    xf = x[:, ::2, :, :3]
    pts_fps = xf.transpose(3, 1, 0, 2).reshape(3, _ROWS, _N)
    anchors = pl.pallas_call(
        _fps_body,
        out_shape=jax.ShapeDtypeStruct((3, _ROWS, _M), jnp.float32),
    )(pts_fps)
    ax = anchors[0].reshape(_ROWS, 1, _M)
    ay = anchors[1].reshape(_ROWS, 1, _M)
    az = anchors[2].reshape(_ROWS, 1, _M)

    fids = [[max(2 * j - 1, 0), 2 * j, min(2 * j + 1, 15)] for j in range(_TP)]
    fids = jnp.asarray(fids, dtype=jnp.int32)
    pts = x[:, fids, :, :3]
    pts = pts.transpose(1, 2, 0, 3, 4).reshape(_NP, _B, _N, 3)

    wg = jnp.concatenate([W_d[:, :3].T, W_f[:, 0][None, :]], axis=0)
    wdelta = (W_pos[:, :3] - W_d[:, :3]).T
    cvec = jnp.stack([W_d[:, 3], W_pos[:, 3], b_pos,
                      jnp.zeros_like(b_pos)], axis=0)

    anch_spec = pl.BlockSpec((1, 1, _M), lambda p, b: ((p // 3) * _B + b, 0, 0))
    g_all, idx = pl.pallas_call(
        _stage2_body,
        grid=(_NP, _B),
        in_specs=[
            pl.BlockSpec((1, 1, _N, 3), lambda p, b: (p, b, 0, 0)),
            anch_spec,
            anch_spec,
            anch_spec,
            pl.BlockSpec((4, _DIM), lambda p, b: (0, 0)),
            pl.BlockSpec((4, _DIM), lambda p, b: (0, 0)),
        ],
        out_specs=[
            pl.BlockSpec((1, 1, _N, _DIM), lambda p, b: (p, b, 0, 0)),
            pl.BlockSpec((1, 1, _M, _K), lambda p, b: (p, b, 0, 0)),
        ],
        out_shape=[
            jax.ShapeDtypeStruct((_NP, _B, _N, _DIM), jnp.float32),
            jax.ShapeDtypeStruct((_NP, _B, _M, _K), jnp.int32),
        ],
    )(pts, ax, ay, az, wg, cvec)

    idx_flat = idx.reshape(_NTASK, _K)
    g_flat = g_all.reshape(_NP * _B * _N, _DIM)

    mesh = plsc.VectorSubcoreMesh(core_axis_name="c", subcore_axis_name="s")
    maxg = pl.kernel(
        _sc_body,
        mesh=mesh,
        out_type=jax.ShapeDtypeStruct((_NTASK, _DIM), jnp.float32),
        scratch_types=[
            pltpu.VMEM((_K,), jnp.int32),
            pltpu.VMEM((_K,), jnp.int32),
            pltpu.VMEM((_K, _DIM), jnp.float32),
            pltpu.VMEM((_K, _DIM), jnp.float32),
            pltpu.VMEM((_DIM,), jnp.float32),
            pltpu.VMEM((_DIM,), jnp.float32),
            pltpu.SemaphoreType.DMA,
            pltpu.SemaphoreType.DMA,
            pltpu.SemaphoreType.DMA,
            pltpu.SemaphoreType.DMA,
        ],
    )(idx_flat, g_flat)

    mx = maxg.reshape(_TP, 3, _B, _M, _DIM)
    epi_anch = pl.BlockSpec((1, 1, _M), lambda j, b: (j * _B + b, 0, 0))
    out = pl.pallas_call(
        _epi_body,
        grid=(_TP, _B),
        in_specs=[
            pl.BlockSpec((1, 3, 1, _M, _DIM), lambda j, b: (j, 0, b, 0, 0)),
            epi_anch,
            epi_anch,
            epi_anch,
            pl.BlockSpec((3, _DIM), lambda j, b: (0, 0)),
            pl.BlockSpec((4, _DIM), lambda j, b: (0, 0)),
        ],
        out_specs=pl.BlockSpec((1, 1, _M, _DIM), lambda j, b: (j, b, 0, 0)),
        out_shape=jax.ShapeDtypeStruct((_TP, _B, _M, _DIM), jnp.float32),
    )(mx, ax, ay, az, wdelta, cvec)

    return out.transpose(1, 0, 2, 3).reshape(_B, _TP * _M, _DIM)
